# baseline (speedup 1.0000x reference)
import functools

import jax
import jax.numpy as jnp
from jax import lax
from jax.experimental import pallas as pl
from jax.experimental.pallas import tpu as pltpu
from jax.experimental.pallas import tpu_sc as plsc

N = 65536
NBUCKETS = 512
NC = 2
NS = 16
NW = NC * NS
PER_W = N // NW
LANES = 16
CHUNKS = PER_W // LANES
SLAB = 8


def _sc_binning_kernel(pred_hbm, dur_hbm, ev_hbm, out_hbm,
                       pred_v, dur_v, ev_v, acc_v, sem):
    wid = lax.axis_index("s") * NC + lax.axis_index("c")
    base = wid * PER_W
    half = PER_W // 2
    cp0 = [pltpu.async_copy(src.at[pl.ds(base, half)], dst.at[pl.ds(0, half)],
                            sem)
           for src, dst in ((pred_hbm, pred_v), (dur_hbm, dur_v),
                            (ev_hbm, ev_v))]
    zeros16 = jnp.zeros((LANES,), jnp.float32)

    @plsc.parallel_loop(0, NBUCKETS // LANES, unroll=4)
    def zero_body(i):
        for q in range(3):
            acc_v[q, pl.ds(i * LANES, LANES)] = zeros16

    for cp in cp0:
        cp.wait()
    cp1 = [pltpu.async_copy(src.at[pl.ds(base + half, half)],
                            dst.at[pl.ds(half, half)], sem)
           for src, dst in ((pred_hbm, pred_v), (dur_hbm, dur_v),
                            (ev_hbm, ev_v))]

    q0 = jnp.zeros((LANES,), jnp.int32)
    q1 = q0 + 1
    q2 = q0 + 2

    def bin_range(lo, hi):
        @plsc.parallel_loop(lo, hi, unroll=8)
        def body(i):
            s = pl.ds(i * LANES, LANES)
            y = pred_v[s]
            d = dur_v[s]
            e = ev_v[s].astype(jnp.float32)
            expy = jnp.exp(jnp.clip(y, -20.0, 20.0))
            plsc.addupdate_scatter(acc_v, [q0, d], expy)
            plsc.addupdate_scatter(acc_v, [q1, d], y * e)
            plsc.addupdate_scatter(acc_v, [q2, d], e)

    bin_range(0, CHUNKS // 2)
    for cp in cp1:
        cp.wait()
    bin_range(CHUNKS // 2, CHUNKS)
    pltpu.sync_copy(acc_v, out_hbm.at[pl.ds(SLAB * wid, SLAB)])


def _sc_binning(pred, durations, events):
    mesh = plsc.VectorSubcoreMesh(core_axis_name="c", subcore_axis_name="s")
    kern = functools.partial(
        pl.kernel,
        mesh=mesh,
        out_type=jax.ShapeDtypeStruct((SLAB * NW, NBUCKETS), jnp.float32),
        scratch_types=[
            pltpu.VMEM((PER_W,), jnp.float32),
            pltpu.VMEM((PER_W,), jnp.int32),
            pltpu.VMEM((PER_W,), jnp.int32),
            pltpu.VMEM((SLAB, NBUCKETS), jnp.float32),
            pltpu.SemaphoreType.DMA,
        ],
        compiler_params=pltpu.CompilerParams(needs_layout_passes=False),
    )(_sc_binning_kernel)
    return kern(pred, durations, events)


def _tc_finalize_kernel(x_ref, o_ref):
    x = x_ref[...]
    rows = x.shape[0]
    q = lax.broadcasted_iota(jnp.int32, (rows, 1), 0) % SLAB
    s_exp = jnp.sum(jnp.where(q == 0, x, 0.0), axis=0, keepdims=True)
    s_ye = jnp.sum(jnp.where(q == 1, x, 0.0), axis=0, keepdims=True)
    s_e = jnp.sum(jnp.where(q == 2, x, 0.0), axis=0, keepdims=True)
    col = lax.broadcasted_iota(jnp.int32, (1, NBUCKETS), 1)
    risk = s_exp
    k = 1
    while k < NBUCKETS:
        shifted = pltpu.roll(risk, NBUCKETS - k, 1)
        risk = risk + jnp.where(col < NBUCKETS - k, shifted, 0.0)
        k *= 2
    ll = s_ye - s_e * jnp.log(jnp.maximum(risk, 1e-12))
    total_ll = jnp.sum(jnp.where(s_e > 0.0, ll, 0.0))
    n_events = jnp.maximum(jnp.sum(s_e), 1.0)
    o_ref[...] = jnp.broadcast_to(-total_ll / n_events, (1, 1))


def kernel(pred, durations, events):
    parts = _sc_binning(pred.reshape(-1).astype(jnp.float32),
                        durations.reshape(-1), events.reshape(-1))
    out = pl.pallas_call(
        _tc_finalize_kernel,
        out_shape=jax.ShapeDtypeStruct((1, 1), jnp.float32),
    )(parts)
    return out.reshape(1)


# speedup vs baseline: 1.0261x; 1.0261x over previous
import functools

import jax
import jax.numpy as jnp
from jax import lax
from jax.experimental import pallas as pl
from jax.experimental.pallas import tpu as pltpu
from jax.experimental.pallas import tpu_sc as plsc

N = 65536
NBUCKETS = 512
NC = 2
NS = 16
NW = NC * NS
PER_W = N // NW
LANES = 16
CHUNKS = PER_W // LANES
SLAB = 8


def _sc_binning_kernel(pred_hbm, dur_hbm, ev_hbm, out_hbm,
                       pred_v, dur_v, ev_v, acc_v, sem):
    wid = lax.axis_index("s") * NC + lax.axis_index("c")
    base = wid * PER_W
    cp_p = pltpu.async_copy(pred_hbm.at[pl.ds(base, PER_W)], pred_v, sem)
    cp_d = pltpu.async_copy(dur_hbm.at[pl.ds(base, PER_W)], dur_v, sem)
    cp_e = pltpu.async_copy(ev_hbm.at[pl.ds(base, PER_W)], ev_v, sem)

    zeros16 = jnp.zeros((LANES,), jnp.float32)

    @plsc.parallel_loop(0, NBUCKETS // LANES, unroll=4)
    def zero_body(i):
        for q in range(3):
            acc_v[q, pl.ds(i * LANES, LANES)] = zeros16

    cp_p.wait()
    cp_d.wait()
    cp_e.wait()

    q0 = jnp.zeros((LANES,), jnp.int32)
    q1 = q0 + 1
    q2 = q0 + 2

    @plsc.parallel_loop(0, CHUNKS, unroll=8)
    def body(i):
        s = pl.ds(i * LANES, LANES)
        y = pred_v[s]
        d = dur_v[s]
        e = ev_v[s].astype(jnp.float32)
        expy = jnp.exp(jnp.clip(y, -20.0, 20.0))
        plsc.addupdate_scatter(acc_v, [q0, d], expy)
        plsc.addupdate_scatter(acc_v, [q1, d], y * e)
        plsc.addupdate_scatter(acc_v, [q2, d], e)

    pltpu.sync_copy(acc_v, out_hbm.at[pl.ds(SLAB * wid, SLAB)])


def _sc_binning(pred, durations, events):
    mesh = plsc.VectorSubcoreMesh(core_axis_name="c", subcore_axis_name="s")
    kern = functools.partial(
        pl.kernel,
        mesh=mesh,
        out_type=jax.ShapeDtypeStruct((SLAB * NW, NBUCKETS), jnp.float32),
        scratch_types=[
            pltpu.VMEM((PER_W,), jnp.float32),
            pltpu.VMEM((PER_W,), jnp.int32),
            pltpu.VMEM((PER_W,), jnp.int32),
            pltpu.VMEM((SLAB, NBUCKETS), jnp.float32),
            pltpu.SemaphoreType.DMA,
        ],
        compiler_params=pltpu.CompilerParams(needs_layout_passes=False),
    )(_sc_binning_kernel)
    return kern(pred, durations, events)


def _tc_finalize_kernel(x_ref, o_ref):
    x = x_ref[...]
    rows = x.shape[0]
    q = lax.broadcasted_iota(jnp.int32, (rows, 1), 0) % SLAB
    s_exp = jnp.sum(jnp.where(q == 0, x, 0.0), axis=0, keepdims=True)
    s_ye = jnp.sum(jnp.where(q == 1, x, 0.0), axis=0, keepdims=True)
    s_e = jnp.sum(jnp.where(q == 2, x, 0.0), axis=0, keepdims=True)
    col = lax.broadcasted_iota(jnp.int32, (1, NBUCKETS), 1)
    risk = s_exp
    k = 1
    while k < NBUCKETS:
        shifted = pltpu.roll(risk, NBUCKETS - k, 1)
        risk = risk + jnp.where(col < NBUCKETS - k, shifted, 0.0)
        k *= 2
    ll = s_ye - s_e * jnp.log(jnp.maximum(risk, 1e-12))
    total_ll = jnp.sum(jnp.where(s_e > 0.0, ll, 0.0))
    n_events = jnp.maximum(jnp.sum(s_e), 1.0)
    o_ref[...] = jnp.broadcast_to(-total_ll / n_events, (1, 1))


def kernel(pred, durations, events):
    parts = _sc_binning(pred.reshape(-1).astype(jnp.float32),
                        durations.reshape(-1), events.reshape(-1))
    out = pl.pallas_call(
        _tc_finalize_kernel,
        out_shape=jax.ShapeDtypeStruct((1, 1), jnp.float32),
    )(parts)
    return out.reshape(1)


# speedup vs baseline: 1.0392x; 1.0128x over previous
import functools

import jax
import jax.numpy as jnp
from jax import lax
from jax.experimental import pallas as pl
from jax.experimental.pallas import tpu as pltpu
from jax.experimental.pallas import tpu_sc as plsc

N = 65536
NBUCKETS = 512
NC = 2
NS = 16
NW = NC * NS
PER_W = N // NW
LANES = 16
CHUNKS = PER_W // LANES
SLAB = 8


def _sc_binning_kernel(pred_hbm, dur_hbm, ev_hbm, out_hbm,
                       pred_v, dur_v, ev_v, acc_v, sem):
    wid = lax.axis_index("s") * NC + lax.axis_index("c")
    base = wid * PER_W
    cp_p = pltpu.async_copy(pred_hbm.at[pl.ds(base, PER_W)], pred_v, sem)
    cp_d = pltpu.async_copy(dur_hbm.at[pl.ds(base, PER_W)], dur_v, sem)
    cp_e = pltpu.async_copy(ev_hbm.at[pl.ds(base, PER_W)], ev_v, sem)

    zeros16 = jnp.zeros((LANES,), jnp.float32)

    @plsc.parallel_loop(0, 3 * NBUCKETS // LANES, unroll=4)
    def zero_body(i):
        acc_v[pl.ds(i * LANES, LANES)] = zeros16

    cp_p.wait()
    cp_d.wait()
    cp_e.wait()

    @plsc.parallel_loop(0, CHUNKS, unroll=8)
    def body(i):
        s = pl.ds(i * LANES, LANES)
        y = pred_v[s]
        d = dur_v[s]
        e = ev_v[s].astype(jnp.float32)
        expy = jnp.exp(jnp.clip(y, -20.0, 20.0))
        plsc.addupdate_scatter(acc_v, [d], expy)
        plsc.addupdate_scatter(acc_v, [d + NBUCKETS], y * e)
        plsc.addupdate_scatter(acc_v, [d + 2 * NBUCKETS], e)

    pltpu.sync_copy(acc_v, out_hbm.at[pl.ds(3 * NBUCKETS * wid, 3 * NBUCKETS)])


def _sc_binning(pred, durations, events):
    mesh = plsc.VectorSubcoreMesh(core_axis_name="c", subcore_axis_name="s")
    kern = functools.partial(
        pl.kernel,
        mesh=mesh,
        out_type=jax.ShapeDtypeStruct((3 * NBUCKETS * NW,), jnp.float32),
        scratch_types=[
            pltpu.VMEM((PER_W,), jnp.float32),
            pltpu.VMEM((PER_W,), jnp.int32),
            pltpu.VMEM((PER_W,), jnp.int32),
            pltpu.VMEM((3 * NBUCKETS,), jnp.float32),
            pltpu.SemaphoreType.DMA,
        ],
        compiler_params=pltpu.CompilerParams(needs_layout_passes=False),
    )(_sc_binning_kernel)
    return kern(pred, durations, events)


def _tc_finalize_kernel(x_ref, o_ref):
    s_exp = jnp.zeros((1, NBUCKETS), jnp.float32)
    s_ye = jnp.zeros((1, NBUCKETS), jnp.float32)
    s_e = jnp.zeros((1, NBUCKETS), jnp.float32)
    for w in range(NW):
        off = 3 * NBUCKETS * w
        s_exp = s_exp + x_ref[pl.ds(off, NBUCKETS)].reshape(1, NBUCKETS)
        s_ye = s_ye + x_ref[pl.ds(off + NBUCKETS, NBUCKETS)].reshape(1, NBUCKETS)
        s_e = s_e + x_ref[pl.ds(off + 2 * NBUCKETS, NBUCKETS)].reshape(1, NBUCKETS)
    col = lax.broadcasted_iota(jnp.int32, (1, NBUCKETS), 1)
    risk = s_exp
    k = 1
    while k < NBUCKETS:
        shifted = pltpu.roll(risk, NBUCKETS - k, 1)
        risk = risk + jnp.where(col < NBUCKETS - k, shifted, 0.0)
        k *= 2
    ll = s_ye - s_e * jnp.log(jnp.maximum(risk, 1e-12))
    total_ll = jnp.sum(jnp.where(s_e > 0.0, ll, 0.0))
    n_events = jnp.maximum(jnp.sum(s_e), 1.0)
    o_ref[...] = jnp.broadcast_to(-total_ll / n_events, (1, 1))


def kernel(pred, durations, events):
    parts = _sc_binning(pred.reshape(-1).astype(jnp.float32),
                        durations.reshape(-1), events.reshape(-1))
    out = pl.pallas_call(
        _tc_finalize_kernel,
        out_shape=jax.ShapeDtypeStruct((1, 1), jnp.float32),
    )(parts)
    return out.reshape(1)
